# baseline (speedup 1.0000x reference)
import functools

import jax
import jax.numpy as jnp
from jax import lax
from jax.experimental import pallas as pl
from jax.experimental.pallas import tpu as pltpu
from jax.experimental.pallas import tpu_sc as plsc

B = 16
MAXLEN = 2048
TOTAL = 16384
D = 512
CU_PAD = 32

NC = 2
NS = 16
NW = NC * NS
RPW = (B * MAXLEN) // NW
WPB = MAXLEN // RPW
LOG2_RPW = RPW.bit_length() - 1

_mesh = plsc.VectorSubcoreMesh(core_axis_name="c", subcore_axis_name="s")


def _body(flat_hbm, cu_hbm, zeros_hbm, out_hbm, cu_v):
    wid = lax.axis_index("s") * NC + lax.axis_index("c")
    b = wid // WPB
    p0 = (wid % WPB) * RPW

    pltpu.sync_copy(cu_hbm, cu_v)
    v0 = cu_v[pl.ds(0, 16)]
    v1 = cu_v[pl.ds(16, 16)]
    iota = lax.iota(jnp.int32, 16)

    def _lane(vec, i):
        return jnp.sum(jnp.where(iota == i, vec, 0))

    cu_b = _lane(v0, b)
    cu_b1 = _lane(v0, b + 1) + _lane(v1, b - 15)

    seg_start = cu_b + p0
    valid = jnp.clip(cu_b1 - cu_b - p0, 0, RPW)

    off = jnp.int32(0)
    for k in range(LOG2_RPW, -1, -1):
        size = 1 << k
        bit = (valid >> k) & 1

        def _copy(off=off, size=size):
            pltpu.sync_copy(
                flat_hbm.at[pl.ds(seg_start + off, size)],
                out_hbm.at[b, pl.ds(p0 + off, size)],
            )

        pl.when(bit == 1)(_copy)
        off = off + bit * size

    pad = RPW - valid
    zoff = valid
    for k in range(LOG2_RPW, -1, -1):
        size = 1 << k
        bit = (pad >> k) & 1

        def _zero(zoff=zoff, size=size):
            pltpu.sync_copy(
                zeros_hbm.at[pl.ds(0, size)],
                out_hbm.at[b, pl.ds(p0 + zoff, size)],
            )

        pl.when(bit == 1)(_zero)
        zoff = zoff + bit * size


_to_dense = functools.partial(
    pl.kernel,
    out_type=jax.ShapeDtypeStruct((B, MAXLEN, D), jnp.float32),
    mesh=_mesh,
    scratch_types=[pltpu.VMEM((CU_PAD,), jnp.int32)],
    compiler_params=pltpu.CompilerParams(
        use_tc_tiling_on_sc=False, needs_layout_passes=False
    ),
)(_body)


def kernel(flat, cu_seqlens):
    cu = jnp.zeros((CU_PAD,), jnp.int32)
    cu = cu.at[: cu_seqlens.shape[0]].set(cu_seqlens.astype(jnp.int32))
    zeros = jnp.zeros((RPW, D), jnp.float32)
    return _to_dense(flat, cu, zeros)
